# Initial kernel scaffold; baseline (speedup 1.0000x reference)
import jax
import jax.numpy as jnp
from jax.experimental import pallas as pl


def kernel(x, edge_index, batch, u_index, emb, Wl, Wr, b, gamma, beta, mlp_W1, mlp_b1, mlp_g, mlp_be, mlp_W2, mlp_b2):
    raise NotImplementedError("write your pallas kernel here")


# speedup vs baseline: 3.8622x; 3.8622x over previous
import functools

import jax
import jax.numpy as jnp
from jax import lax
from jax.experimental import pallas as pl
from jax.experimental.pallas import tpu as pltpu
from jax.experimental.pallas import tpu_sc as plsc

_EPS = 1e-5
_NC, _NS = 2, 16
_NW = _NC * _NS
_CH = 128


def _cdiv(a, b):
    return (a + b - 1) // b


def _sc_gather_rows(table, idx_pad, H):
    n_pad = idx_pad.shape[0]
    k_per_w = n_pad // (_NW * _CH)
    mesh = plsc.VectorSubcoreMesh(core_axis_name="c", subcore_axis_name="s")

    @functools.partial(
        pl.kernel,
        out_type=jax.ShapeDtypeStruct((n_pad, H), jnp.float32),
        mesh=mesh,
        scratch_types=[
            pltpu.VMEM((_CH,), jnp.int32),
            pltpu.VMEM((_CH, H), jnp.float32),
            pltpu.SemaphoreType.DMA,
        ],
    )
    def k(tab_hbm, idx_hbm, out_hbm, idx, rows, sem):
        cid = lax.axis_index("c")
        sid = lax.axis_index("s")
        wid = sid * _NC + cid
        base = wid * (k_per_w * _CH)

        def body(j, carry):
            off = base + j * _CH
            pltpu.sync_copy(idx_hbm.at[pl.ds(off, _CH)], idx)
            pltpu.async_copy(tab_hbm.at[idx], rows, sem).wait()
            pltpu.sync_copy(rows, out_hbm.at[pl.ds(off, _CH)])
            return carry

        lax.fori_loop(0, k_per_w, body, 0)

    return k(table, idx_pad)


def _sc_segment_sum(h, srcp, dstp, n_pad):
    H = h.shape[1]
    e_pad = srcp.shape[0]
    k_per_w = e_pad // (_NW * _CH)
    rpt = n_pad // _NS
    mesh = plsc.VectorSubcoreMesh(core_axis_name="c", subcore_axis_name="s")

    @functools.partial(
        pl.kernel,
        out_type=jax.ShapeDtypeStruct((2 * n_pad, H), jnp.float32),
        mesh=mesh,
        scratch_types=[
            pltpu.VMEM((_CH,), jnp.int32),
            pltpu.VMEM((_CH,), jnp.int32),
            pltpu.VMEM((_CH, H), jnp.float32),
            pltpu.VMEM_SHARED((n_pad, H), jnp.float32),
            pltpu.SemaphoreType.DMA,
        ],
    )
    def k(h_hbm, src_hbm, dst_hbm, out_hbm, sidx, didx, rows, acc, sem):
        cid = lax.axis_index("c")
        sid = lax.axis_index("s")
        wid = sid * _NC + cid
        ebase = wid * (k_per_w * _CH)

        def zrow(i, carry):
            for t in range(H // 16):
                rows[i, pl.ds(t * 16, 16)] = jnp.zeros((16,), jnp.float32)
            return carry

        lax.fori_loop(0, _CH, zrow, 0)

        def zcp(i, carry):
            pltpu.sync_copy(rows, acc.at[pl.ds(sid * rpt + i * _CH, _CH)])
            return carry

        lax.fori_loop(0, rpt // _CH, zcp, 0)
        plsc.subcore_barrier()

        def body(j, carry):
            off = ebase + j * _CH
            pltpu.sync_copy(src_hbm.at[pl.ds(off, _CH)], sidx)
            pltpu.sync_copy(dst_hbm.at[pl.ds(off, _CH)], didx)
            pltpu.async_copy(h_hbm.at[sidx], rows, sem).wait()
            pltpu.sync_copy(rows, acc.at[didx], add=True)
            return carry

        lax.fori_loop(0, k_per_w, body, 0)
        plsc.subcore_barrier()
        pltpu.sync_copy(
            acc.at[pl.ds(sid * rpt, rpt)],
            out_hbm.at[pl.ds(cid * n_pad + sid * rpt, rpt)],
        )

    return k(h, srcp, dstp)


def _sc_degree(dstp, n_pad, H):
    e_pad = dstp.shape[0]
    k_per_w = e_pad // (_NW * _CH)
    rpt = n_pad // _NS
    mesh = plsc.VectorSubcoreMesh(core_axis_name="c", subcore_axis_name="s")

    @functools.partial(
        pl.kernel,
        out_type=jax.ShapeDtypeStruct((2 * n_pad, H), jnp.float32),
        mesh=mesh,
        scratch_types=[
            pltpu.VMEM((_CH,), jnp.int32),
            pltpu.VMEM((_CH, H), jnp.float32),
            pltpu.VMEM_SHARED((n_pad, H), jnp.float32),
            pltpu.SemaphoreType.DMA,
        ],
    )
    def k(dst_hbm, out_hbm, didx, ones, acc, sem):
        cid = lax.axis_index("c")
        sid = lax.axis_index("s")
        wid = sid * _NC + cid
        ebase = wid * (k_per_w * _CH)

        def zrow(i, carry):
            for t in range(H // 16):
                ones[i, pl.ds(t * 16, 16)] = jnp.zeros((16,), jnp.float32)
            return carry

        lax.fori_loop(0, _CH, zrow, 0)

        def zcp(i, carry):
            pltpu.sync_copy(ones, acc.at[pl.ds(sid * rpt + i * _CH, _CH)])
            return carry

        lax.fori_loop(0, rpt // _CH, zcp, 0)

        def orow(i, carry):
            for t in range(H // 16):
                ones[i, pl.ds(t * 16, 16)] = jnp.ones((16,), jnp.float32)
            return carry

        lax.fori_loop(0, _CH, orow, 0)
        plsc.subcore_barrier()

        def body(j, carry):
            off = ebase + j * _CH
            pltpu.sync_copy(dst_hbm.at[pl.ds(off, _CH)], didx)
            pltpu.sync_copy(ones, acc.at[didx], add=True)
            return carry

        lax.fori_loop(0, k_per_w, body, 0)
        plsc.subcore_barrier()
        pltpu.sync_copy(
            acc.at[pl.ds(sid * rpt, rpt)],
            out_hbm.at[pl.ds(cid * n_pad + sid * rpt, rpt)],
        )

    return k(dstp)


def _tc_pre(part, degp, h, wl, wr, bl, N, B, H):
    grid = N // B

    def body(p_ref, d_ref, h_ref, wl_ref, wr_ref, b_ref, z_ref, s_ref, s2_ref):
        i = pl.program_id(0)
        deg = d_ref[0, :, 0:1] + d_ref[1, :, 0:1]
        rdeg = 1.0 / jnp.maximum(deg, 1.0)
        agg = (p_ref[0] + p_ref[1]) * rdeg
        z = (jnp.dot(agg, wl_ref[...], preferred_element_type=jnp.float32)
             + jnp.dot(h_ref[...], wr_ref[...], preferred_element_type=jnp.float32)
             + b_ref[...])
        z_ref[...] = z

        @pl.when(i == 0)
        def _():
            s_ref[...] = jnp.zeros_like(s_ref)
            s2_ref[...] = jnp.zeros_like(s2_ref)

        s_ref[...] += jnp.sum(z, axis=0, keepdims=True)
        s2_ref[...] += jnp.sum(z * z, axis=0, keepdims=True)

    return pl.pallas_call(
        body,
        grid=(grid,),
        in_specs=[
            pl.BlockSpec((2, B, H), lambda i: (0, i, 0)),
            pl.BlockSpec((2, B, H), lambda i: (0, i, 0)),
            pl.BlockSpec((B, H), lambda i: (i, 0)),
            pl.BlockSpec((H, H), lambda i: (0, 0)),
            pl.BlockSpec((H, H), lambda i: (0, 0)),
            pl.BlockSpec((1, H), lambda i: (0, 0)),
        ],
        out_specs=[
            pl.BlockSpec((B, H), lambda i: (i, 0)),
            pl.BlockSpec((1, H), lambda i: (0, 0)),
            pl.BlockSpec((1, H), lambda i: (0, 0)),
        ],
        out_shape=[
            jax.ShapeDtypeStruct((N, H), jnp.float32),
            jax.ShapeDtypeStruct((1, H), jnp.float32),
            jax.ShapeDtypeStruct((1, H), jnp.float32),
        ],
    )(part, degp, h, wl, wr, bl)


def _tc_post(z, s, s2, g, be, N, B, H):
    grid = N // B

    def body(z_ref, s_ref, s2_ref, g_ref, b_ref, h_ref):
        m = s_ref[...] * (1.0 / N)
        v = s2_ref[...] * (1.0 / N) - m * m
        inv = lax.rsqrt(v + _EPS)
        h_ref[...] = jnp.maximum(
            (z_ref[...] - m) * inv * g_ref[...] + b_ref[...], 0.0)

    return pl.pallas_call(
        body,
        grid=(grid,),
        in_specs=[
            pl.BlockSpec((B, H), lambda i: (i, 0)),
            pl.BlockSpec((1, H), lambda i: (0, 0)),
            pl.BlockSpec((1, H), lambda i: (0, 0)),
            pl.BlockSpec((1, H), lambda i: (0, 0)),
            pl.BlockSpec((1, H), lambda i: (0, 0)),
        ],
        out_specs=pl.BlockSpec((B, H), lambda i: (i, 0)),
        out_shape=jax.ShapeDtypeStruct((N, H), jnp.float32),
    )(z, s, s2, g, be)


def _tc_post_pool(z, s, s2, g, be, batch3, N, B, H, G):
    grid = N // B

    def body(z_ref, s_ref, s2_ref, g_ref, b_ref, bt_ref, out_ref, acc, cacc):
        i = pl.program_id(0)
        m = s_ref[...] * (1.0 / N)
        v = s2_ref[...] * (1.0 / N) - m * m
        inv = lax.rsqrt(v + _EPS)
        h = jnp.maximum((z_ref[...] - m) * inv * g_ref[...] + b_ref[...], 0.0)
        gids = lax.broadcasted_iota(jnp.int32, (G, B), 0)
        oh = (bt_ref[0] == gids).astype(jnp.float32)

        @pl.when(i == 0)
        def _():
            acc[...] = jnp.zeros_like(acc)
            cacc[...] = jnp.zeros_like(cacc)

        acc[...] += jnp.dot(oh, h, preferred_element_type=jnp.float32)
        cacc[...] += jnp.sum(oh, axis=1, keepdims=True)

        @pl.when(i == grid - 1)
        def _():
            out_ref[...] = acc[...] / jnp.maximum(cacc[...], 1.0)

    return pl.pallas_call(
        body,
        grid=(grid,),
        in_specs=[
            pl.BlockSpec((B, H), lambda i: (i, 0)),
            pl.BlockSpec((1, H), lambda i: (0, 0)),
            pl.BlockSpec((1, H), lambda i: (0, 0)),
            pl.BlockSpec((1, H), lambda i: (0, 0)),
            pl.BlockSpec((1, H), lambda i: (0, 0)),
            pl.BlockSpec((1, 1, B), lambda i: (i, 0, 0)),
        ],
        out_specs=pl.BlockSpec((G, H), lambda i: (0, 0)),
        out_shape=jax.ShapeDtypeStruct((G, H), jnp.float32),
        scratch_shapes=[
            pltpu.VMEM((G, H), jnp.float32),
            pltpu.VMEM((G, 1), jnp.float32),
        ],
    )(z, s, s2, g, be, batch3)


def kernel(x, edge_index, batch, u_index, emb, Wl, Wr, b, gamma, beta,
           mlp_W1, mlp_b1, mlp_g, mlp_be, mlp_W2, mlp_b2):
    N = x.shape[0]
    E = edge_index.shape[1]
    H = emb.shape[1]
    L = Wl.shape[0]
    G = 16
    B = 1000

    n_pad = _NS * _CH * _cdiv(N + 1, _NS * _CH)
    e_pad = _NW * _CH * _cdiv(E, _NW * _CH)
    x_pad = _NW * _CH * _cdiv(N, _NW * _CH)

    src = edge_index[0]
    dst = edge_index[1]
    srcp = jnp.concatenate([src, jnp.zeros((e_pad - E,), src.dtype)])
    dstp = jnp.concatenate([dst, jnp.full((e_pad - E,), N, dst.dtype)])
    xp = jnp.concatenate([x, jnp.zeros((x_pad - N,), x.dtype)])

    h = _sc_gather_rows(emb, xp, H)[:N]
    degp = _sc_degree(dstp, n_pad, H).reshape(2, n_pad, H)
    batch3 = batch.reshape(N // B, 1, B)

    pooled = None
    for l in range(L):
        part = _sc_segment_sum(h, srcp, dstp, n_pad).reshape(2, n_pad, H)
        z, s, s2 = _tc_pre(part, degp, h, Wl[l], Wr[l], b[l].reshape(1, H),
                           N, B, H)
        gl = gamma[l].reshape(1, H)
        bl = beta[l].reshape(1, H)
        if l < L - 1:
            h = _tc_post(z, s, s2, gl, bl, N, B, H)
        else:
            pooled = _tc_post_pool(z, s, s2, gl, bl, batch3, N, B, H, G)
    return pooled
